# baseline (speedup 1.0000x reference)
import functools

import jax
import jax.numpy as jnp
import numpy as np
from jax import lax
from jax.experimental import pallas as pl
from jax.experimental.pallas import tpu as pltpu
from jax.experimental.pallas import tpu_sc as plsc

_BASE_AMP = 0.08
_LANES = 16


def _anchor_starts(signal_len, num_segments, seg_length):
    max_start = max(signal_len - seg_length, 0)
    head = 0.1 * signal_len
    tail = max(0.0, 0.78 * signal_len)
    anchors = np.linspace(head, tail, num_segments)
    return np.clip(np.round(anchors), 0, max_start).astype(np.int64)


def _sc_trig_builder(starts, seg_len, sig_len, nseg, pad):
    S = sig_len
    mesh = plsc.VectorSubcoreMesh(core_axis_name="c", subcore_axis_name="s")

    @functools.partial(
        pl.kernel,
        mesh=mesh,
        out_type=jax.ShapeDtypeStruct((2, S), jnp.float32),
        scratch_types=[
            pltpu.VMEM((nseg, pad), jnp.float32),
            pltpu.VMEM((nseg, pad), jnp.float32),
            pltpu.VMEM((nseg, _LANES), jnp.float32),
            pltpu.VMEM((2, S), jnp.float32),
        ],
    )
    def trig_kernel(pi_hbm, pq_hbm, sc_hbm, trig_hbm, tpi_v, tpq_v, scv, trig_v):
        wid = lax.axis_index("s") * 2 + lax.axis_index("c")

        @pl.when(wid == 0)
        def _():
            pltpu.sync_copy(pi_hbm, tpi_v)
            pltpu.sync_copy(pq_hbm, tpq_v)
            pltpu.sync_copy(sc_hbm, scv)

            for k in range(nseg):
                def tanh_chunk(j, _, k=k):
                    for ref in (tpi_v, tpq_v):
                        z = ref[k, pl.ds(j * _LANES, _LANES)]
                        e = jnp.exp(2.0 * z)
                        ref[k, pl.ds(j * _LANES, _LANES)] = 1.0 - 2.0 / (e + 1.0)
                    return 0

                lax.fori_loop(0, pad // _LANES, tanh_chunk, 0)

            zeros = jnp.zeros((_LANES,), jnp.float32)

            def zero_chunk(j, _):
                trig_v[0, pl.ds(j * _LANES, _LANES)] = zeros
                trig_v[1, pl.ds(j * _LANES, _LANES)] = zeros
                return 0

            lax.fori_loop(0, S // _LANES, zero_chunk, 0)

            for k, s in enumerate(starts):
                g = jnp.maximum(scv[k], 0.0)
                m = (s // _LANES)
                nch = min(pad, S - m * _LANES) // _LANES

                def seg_chunk(j, _, k=k, g=g, m=m):
                    trig_v[0, pl.ds((m + j) * _LANES, _LANES)] += (
                        g * tpi_v[k, pl.ds(j * _LANES, _LANES)])
                    trig_v[1, pl.ds((m + j) * _LANES, _LANES)] += (
                        g * tpq_v[k, pl.ds(j * _LANES, _LANES)])
                    return 0

                lax.fori_loop(0, nch, seg_chunk, 0)

            pltpu.sync_copy(trig_v, trig_hbm)

    return trig_kernel


def _tc_body(x_ref, trig_ref, o_ref, *, sig_len):
    xv = x_ref[...]
    ssq = jnp.sum(xv * xv, axis=(1, 2), keepdims=True)
    amp = _BASE_AMP * jnp.sqrt(ssq / (2.0 * sig_len) + 1e-12)
    o_ref[...] = xv + amp * trig_ref[...][None, :, :]


def kernel(x, pattern_i, pattern_q, segment_scale):
    batch, ch, S = x.shape
    L = pattern_i.shape[0]
    nseg = segment_scale.shape[0]
    starts = tuple(int(v) for v in _anchor_starts(S, nseg, L))

    pad = L + _LANES
    pi_rows = [jnp.pad(pattern_i, (s % _LANES, pad - L - s % _LANES))
               for s in starts]
    pq_rows = [jnp.pad(pattern_q, (s % _LANES, pad - L - s % _LANES))
               for s in starts]
    pi_sh = jnp.stack(pi_rows)
    pq_sh = jnp.stack(pq_rows)
    scales_b = jnp.tile(segment_scale[:, None], (1, _LANES))

    trig = _sc_trig_builder(starts, L, S, nseg, pad)(pi_sh, pq_sh, scales_b)

    TB = 128
    body = functools.partial(_tc_body, sig_len=S)
    out = pl.pallas_call(
        body,
        grid=(batch // TB,),
        in_specs=[
            pl.BlockSpec((TB, ch, S), lambda i: (i, 0, 0)),
            pl.BlockSpec((ch, S), lambda i: (0, 0)),
        ],
        out_specs=pl.BlockSpec((TB, ch, S), lambda i: (i, 0, 0)),
        out_shape=jax.ShapeDtypeStruct((batch, ch, S), jnp.float32),
        compiler_params=pltpu.CompilerParams(
            dimension_semantics=("parallel",)),
    )(x, trig)
    return out


# speedup vs baseline: 1.0145x; 1.0145x over previous
import functools

import jax
import jax.numpy as jnp
import numpy as np
from jax import lax
from jax.experimental import pallas as pl
from jax.experimental.pallas import tpu as pltpu
from jax.experimental.pallas import tpu_sc as plsc

_BASE_AMP = 0.08
_LANES = 16


def _anchor_starts(signal_len, num_segments, seg_length):
    max_start = max(signal_len - seg_length, 0)
    head = 0.1 * signal_len
    tail = max(0.0, 0.78 * signal_len)
    anchors = np.linspace(head, tail, num_segments)
    return np.clip(np.round(anchors), 0, max_start).astype(np.int64)


def _sc_trig_builder(starts, seg_len, sig_len, nseg, pad):
    S = sig_len
    NW = 32
    RANGE = 2 * S // NW
    per_ch = RANGE
    mesh = plsc.VectorSubcoreMesh(core_axis_name="c", subcore_axis_name="s")

    plan = []
    for w in range(NW):
        c = w // (NW // 2)
        r0 = (w % (NW // 2)) * per_ch
        r1 = r0 + per_ch
        jobs = []
        for k, s in enumerate(starts):
            m16 = (s // _LANES) * _LANES
            o0 = max(r0, m16)
            o1 = min(r1, min(m16 + pad, S))
            if o0 < o1:
                jobs.append((k, (o0 - r0) // _LANES, (o0 - m16) // _LANES,
                             (o1 - o0) // _LANES))
        plan.append((c, r0, jobs))
    max_rows = max(1, max(len(jobs) for _, _, jobs in plan))

    @functools.partial(
        pl.kernel,
        mesh=mesh,
        out_type=jax.ShapeDtypeStruct((2, S), jnp.float32),
        scratch_types=[
            pltpu.VMEM((max_rows, pad), jnp.float32),
            pltpu.VMEM((nseg, _LANES), jnp.float32),
            pltpu.VMEM((RANGE,), jnp.float32),
        ],
    )
    def trig_kernel(pi_hbm, pq_hbm, sc_hbm, trig_hbm, stg_v, scv, loc_v):
        wid = lax.axis_index("s") * 2 + lax.axis_index("c")

        for w, (c, r0, jobs) in enumerate(plan):
            @pl.when(wid == w)
            def _(c=c, r0=r0, jobs=jobs):
                pltpu.sync_copy(sc_hbm, scv)
                src = pi_hbm if c == 0 else pq_hbm
                for slot, (k, _, _, _) in enumerate(jobs):
                    pltpu.sync_copy(src.at[k], stg_v.at[slot])

                for slot in range(len(jobs)):
                    def tanh_chunk(j, _, slot=slot):
                        z = stg_v[slot, pl.ds(j * _LANES, _LANES)]
                        e = jnp.exp(2.0 * z)
                        stg_v[slot, pl.ds(j * _LANES, _LANES)] = (
                            1.0 - 2.0 / (e + 1.0))
                        return 0

                    lax.fori_loop(0, pad // _LANES, tanh_chunk, 0)

                zeros = jnp.zeros((_LANES,), jnp.float32)

                def zero_chunk(j, _):
                    loc_v[pl.ds(j * _LANES, _LANES)] = zeros
                    return 0

                lax.fori_loop(0, per_ch // _LANES, zero_chunk, 0)

                for slot, (k, dloc, dstg, nch) in enumerate(jobs):
                    g = jnp.maximum(scv[k], 0.0)

                    def seg_chunk(j, _, slot=slot, g=g, dloc=dloc, dstg=dstg):
                        loc_v[pl.ds((dloc + j) * _LANES, _LANES)] += (
                            g * stg_v[slot, pl.ds((dstg + j) * _LANES, _LANES)])
                        return 0

                    lax.fori_loop(0, nch, seg_chunk, 0)

                pltpu.sync_copy(loc_v, trig_hbm.at[c, pl.ds(r0, per_ch)])

    return trig_kernel


def _tc_body(x_ref, trig_ref, o_ref, *, sig_len):
    xv = x_ref[...]
    ssq = jnp.sum(xv * xv, axis=(1, 2), keepdims=True)
    amp = _BASE_AMP * jnp.sqrt(ssq / (2.0 * sig_len) + 1e-12)
    o_ref[...] = xv + amp * trig_ref[...][None, :, :]


def kernel(x, pattern_i, pattern_q, segment_scale):
    batch, ch, S = x.shape
    L = pattern_i.shape[0]
    nseg = segment_scale.shape[0]
    starts = tuple(int(v) for v in _anchor_starts(S, nseg, L))

    pad = L + _LANES
    pi_rows = [jnp.pad(pattern_i, (s % _LANES, pad - L - s % _LANES))
               for s in starts]
    pq_rows = [jnp.pad(pattern_q, (s % _LANES, pad - L - s % _LANES))
               for s in starts]
    pi_sh = jnp.stack(pi_rows)
    pq_sh = jnp.stack(pq_rows)
    scales_b = jnp.tile(segment_scale[:, None], (1, _LANES))

    trig = _sc_trig_builder(starts, L, S, nseg, pad)(pi_sh, pq_sh, scales_b)

    TB = 128
    body = functools.partial(_tc_body, sig_len=S)
    out = pl.pallas_call(
        body,
        grid=(batch // TB,),
        in_specs=[
            pl.BlockSpec((TB, ch, S), lambda i: (i, 0, 0)),
            pl.BlockSpec((ch, S), lambda i: (0, 0)),
        ],
        out_specs=pl.BlockSpec((TB, ch, S), lambda i: (i, 0, 0)),
        out_shape=jax.ShapeDtypeStruct((batch, ch, S), jnp.float32),
        compiler_params=pltpu.CompilerParams(
            dimension_semantics=("parallel",)),
    )(x, trig)
    return out


# speedup vs baseline: 1.5744x; 1.5518x over previous
import functools

import jax
import jax.numpy as jnp
import numpy as np
from jax.experimental import pallas as pl
from jax.experimental.pallas import tpu as pltpu

_BASE_AMP = 0.08


def _anchor_starts(signal_len, num_segments, seg_length):
    max_start = max(signal_len - seg_length, 0)
    head = 0.1 * signal_len
    tail = max(0.0, 0.78 * signal_len)
    anchors = np.linspace(head, tail, num_segments)
    return np.clip(np.round(anchors), 0, max_start).astype(np.int64)


def _body(x_ref, pi_ref, pq_ref, sc_ref, o_ref, trig_ref, *,
          starts, seg_len, sig_len):
    S = sig_len
    pat_i = jnp.tanh(pi_ref[...])
    pat_q = jnp.tanh(pq_ref[...])
    trig_ref[...] = jnp.zeros(trig_ref.shape, jnp.float32)
    for k, s in enumerate(starts):
        e = min(s + seg_len, S)
        L = e - s
        g = jnp.maximum(sc_ref[k], 0.0)
        trig_ref[0:1, pl.ds(s, L)] += g * pat_i[:, :L]
        trig_ref[1:2, pl.ds(s, L)] += g * pat_q[:, :L]
    xv = x_ref[...]
    ssq = jnp.sum(xv * xv, axis=(1, 2), keepdims=True)
    amp = _BASE_AMP * jnp.sqrt(ssq / (2.0 * S) + 1e-12)
    o_ref[...] = xv + amp * trig_ref[...][None, :, :]


def kernel(x, pattern_i, pattern_q, segment_scale):
    batch, ch, S = x.shape
    L = pattern_i.shape[0]
    nseg = segment_scale.shape[0]
    starts = tuple(int(v) for v in _anchor_starts(S, nseg, L))

    TB = 128
    body = functools.partial(_body, starts=starts, seg_len=L, sig_len=S)
    out = pl.pallas_call(
        body,
        grid=(batch // TB,),
        in_specs=[
            pl.BlockSpec((TB, ch, S), lambda i: (i, 0, 0)),
            pl.BlockSpec((1, L), lambda i: (0, 0)),
            pl.BlockSpec((1, L), lambda i: (0, 0)),
            pl.BlockSpec(memory_space=pltpu.SMEM),
        ],
        out_specs=pl.BlockSpec((TB, ch, S), lambda i: (i, 0, 0)),
        out_shape=jax.ShapeDtypeStruct((batch, ch, S), jnp.float32),
        scratch_shapes=[pltpu.VMEM((ch, S), jnp.float32)],
        compiler_params=pltpu.CompilerParams(
            dimension_semantics=("parallel",)),
    )(x, pattern_i.reshape(1, L), pattern_q.reshape(1, L), segment_scale)
    return out
